# Initial kernel scaffold; baseline (speedup 1.0000x reference)
import functools

import jax
import jax.numpy as jnp
from jax import lax
from jax.experimental import pallas as pl
from jax.experimental.pallas import tpu as pltpu
from jax.experimental.pallas import tpu_sc as plsc

N = 50000
NPAD = 50048
E = 800000
EPAD = 819200
G = 1024
GACC = 1040
NCORES = 2
NSUB = 16
NW = NCORES * NSUB

IDXROWS = EPAD // 128
IDXROWS_PER_W = IDXROWS // NW
SUBROWS = 8
CHUNK = SUBROWS * 128
NCHUNKS = IDXROWS_PER_W // SUBROWS
RPT = NPAD // NSUB
ZROWS = RPT // 4

HROWS = NPAD // 128
BN = 544
NBLK = NPAD // BN


def _mesh():
    return plsc.VectorSubcoreMesh(
        core_axis_name="c", subcore_axis_name="s",
        num_cores=NCORES, num_subcores=NSUB)


@functools.lru_cache(maxsize=None)
def _make_edge_scatter(C):
    vpr = C // 16

    @functools.partial(
        pl.kernel,
        out_type=jax.ShapeDtypeStruct((NCORES * NPAD, C), jnp.float32),
        mesh=_mesh(),
        scratch_types=[
            pltpu.VMEM((SUBROWS, 128), jnp.int32),
            pltpu.VMEM((SUBROWS, 128), jnp.int32),
            pltpu.VMEM((SUBROWS, 128), jnp.float32),
            pltpu.VMEM((CHUNK, C), jnp.float32),
            pltpu.VMEM((ZROWS, C), jnp.float32),
            pltpu.VMEM_SHARED((NPAD, C), jnp.float32),
            pltpu.SemaphoreType.DMA,
        ],
    )
    def edge_scatter(z_hbm, src_hbm, dst_hbm, ew_hbm, out_hbm,
                     src_v, dst_v, ew_v, rows_v, zero_v, acc_sh, sem):
        cid = lax.axis_index("c")
        sid = lax.axis_index("s")
        wid = cid * NSUB + sid

        @pl.loop(0, ZROWS)
        def _zero_fill(r):
            for v in range(vpr):
                zero_v[r, pl.ds(16 * v, 16)] = jnp.zeros((16,), jnp.float32)

        for q in range(4):
            pltpu.sync_copy(
                zero_v, acc_sh.at[pl.ds(sid * RPT + q * ZROWS, ZROWS)])
        plsc.subcore_barrier()

        base_row = wid * IDXROWS_PER_W

        @pl.loop(0, NCHUNKS)
        def _chunk(gidx):
            r0 = base_row + gidx * SUBROWS
            pltpu.sync_copy(src_hbm.at[pl.ds(r0, SUBROWS)], src_v)
            pltpu.sync_copy(dst_hbm.at[pl.ds(r0, SUBROWS)], dst_v)
            pltpu.sync_copy(ew_hbm.at[pl.ds(r0, SUBROWS)], ew_v)
            gathers = [
                pltpu.async_copy(z_hbm.at[src_v.at[j]],
                                 rows_v.at[pl.ds(j * 128, 128)], sem)
                for j in range(SUBROWS)
            ]
            for cp in gathers:
                cp.wait()

            @pl.loop(0, CHUNK, unroll=4)
            def _scale(e):
                j = e >> 7
                l = e & 127
                w = plsc.load_gather(
                    ew_v,
                    [jnp.full((16,), j, jnp.int32),
                     jnp.full((16,), l, jnp.int32)])
                for v in range(vpr):
                    rows_v[e, pl.ds(16 * v, 16)] = (
                        rows_v[e, pl.ds(16 * v, 16)] * w)

            scatters = [
                pltpu.async_copy(rows_v.at[pl.ds(j * 128, 128)],
                                 acc_sh.at[dst_v.at[j]], sem, add=True)
                for j in range(SUBROWS)
            ]
            for cp in scatters:
                cp.wait()

        plsc.subcore_barrier()
        pltpu.sync_copy(
            acc_sh.at[pl.ds(sid * RPT, RPT)],
            out_hbm.at[pl.ds(cid * NPAD + sid * RPT, RPT)])

    return edge_scatter


@functools.partial(
    pl.kernel,
    out_type=(jax.ShapeDtypeStruct((NCORES * G, 128), jnp.float32),
              jax.ShapeDtypeStruct((NCORES * G, 16), jnp.float32)),
    mesh=_mesh(),
    scratch_types=[
        pltpu.VMEM((1, 128), jnp.int32),
        pltpu.VMEM((128, 128), jnp.float32),
        pltpu.VMEM((128, 16), jnp.float32),
        pltpu.VMEM((65, 128), jnp.float32),
        pltpu.VMEM((65, 16), jnp.float32),
        pltpu.VMEM_SHARED((GACC, 128), jnp.float32),
        pltpu.VMEM_SHARED((GACC, 16), jnp.float32),
    ],
)
def _pool_kernel(h_hbm, b_hbm, sum_hbm, cnt_hbm,
                 bidx_v, rows_v, ones_v, zs_v, zc_v, accs_sh, accc_sh):
    cid = lax.axis_index("c")
    sid = lax.axis_index("s")
    wid = cid * NSUB + sid

    @pl.loop(0, 65)
    def _fill_z(r):
        for v in range(8):
            zs_v[r, pl.ds(16 * v, 16)] = jnp.zeros((16,), jnp.float32)
        zc_v[r, pl.ds(0, 16)] = jnp.zeros((16,), jnp.float32)

    @pl.loop(0, 128)
    def _fill_one(r):
        ones_v[r, pl.ds(0, 16)] = jnp.ones((16,), jnp.float32)

    pltpu.sync_copy(zs_v, accs_sh.at[pl.ds(sid * 65, 65)])
    pltpu.sync_copy(zc_v, accc_sh.at[pl.ds(sid * 65, 65)])
    plsc.subcore_barrier()

    extra = jnp.where(wid < 7, 1, 0)
    base = wid * 12 + jnp.minimum(wid, 7)

    @pl.loop(0, 12 + extra)
    def _row(jj):
        j = base + jj
        pltpu.sync_copy(h_hbm.at[pl.ds(j * 128, 128)], rows_v)
        pltpu.sync_copy(b_hbm.at[pl.ds(j, 1)], bidx_v)
        pltpu.sync_copy(rows_v, accs_sh.at[bidx_v.at[0]], add=True)
        pltpu.sync_copy(ones_v, accc_sh.at[bidx_v.at[0]], add=True)

    plsc.subcore_barrier()
    pltpu.sync_copy(accs_sh.at[pl.ds(sid * 64, 64)],
                    sum_hbm.at[pl.ds(cid * G + sid * 64, 64)])
    pltpu.sync_copy(accc_sh.at[pl.ds(sid * 64, 64)],
                    cnt_hbm.at[pl.ds(cid * G + sid * 64, 64)])


def _tc_layer(parts, W, g, be, split=False):
    widths = [p.shape[1] for p in parts]
    cin = sum(widths)
    dout = W.shape[1]
    nparts = len(parts)

    def body(*refs):
        a_refs = refs[:2 * nparts]
        w_ref, g_ref, be_ref = refs[2 * nparts:2 * nparts + 3]
        if split:
            outa, outb, sacc, c2acc = refs[2 * nparts + 3:]
        else:
            out, sacc, c2acc = refs[2 * nparts + 3:]
        t = pl.program_id(0)
        i = pl.program_id(1)

        zs = []
        for p in range(nparts):
            zs.append(a_refs[2 * p][0] + a_refs[2 * p + 1][0])
        z = zs[0] if nparts == 1 else jnp.concatenate(zs, axis=1)

        @pl.when(jnp.logical_and(t == 0, i == 0))
        def _init():
            sacc[...] = jnp.zeros_like(sacc)
            c2acc[...] = jnp.zeros_like(c2acc)

        @pl.when(t == 0)
        def _stats():
            sacc[...] += jnp.sum(z, axis=0, keepdims=True)
            c2acc[...] += lax.dot_general(
                z, z, (((0,), (0,)), ((), ())),
                preferred_element_type=jnp.float32)

        @pl.when(t == 1)
        def _emit():
            wmat = w_ref[...]
            m = sacc[...] * (1.0 / N)
            my = jnp.dot(m, wmat, preferred_element_type=jnp.float32)
            p2 = jnp.dot(c2acc[...] * (1.0 / N), wmat,
                         preferred_element_type=jnp.float32)
            ey2 = jnp.sum(wmat * p2, axis=0, keepdims=True)
            var = ey2 - my * my
            scale = g_ref[...] * lax.rsqrt(var + 1e-5)
            shift = be_ref[...] - my * scale
            y = jnp.dot(z, wmat,
                        preferred_element_type=jnp.float32) * scale + shift
            y = jnp.where(y >= 0, y, 0.01 * y)
            if split:
                outa[...] = y[:, :dout // 2]
                outb[...] = y[:, dout // 2:]
            else:
                out[...] = y

    in_specs = []
    for p, cp in enumerate(widths):
        for c in range(NCORES):
            in_specs.append(pl.BlockSpec(
                (1, BN, cp), lambda t, i, c=c: (c, i, 0)))
    in_specs += [
        pl.BlockSpec((cin, dout), lambda t, i: (0, 0)),
        pl.BlockSpec((1, dout), lambda t, i: (0, 0)),
        pl.BlockSpec((1, dout), lambda t, i: (0, 0)),
    ]
    if split:
        out_shape = (jax.ShapeDtypeStruct((NPAD, dout // 2), jnp.float32),
                     jax.ShapeDtypeStruct((NPAD, dout // 2), jnp.float32))
        out_specs = (pl.BlockSpec((BN, dout // 2), lambda t, i: (i, 0)),
                     pl.BlockSpec((BN, dout // 2), lambda t, i: (i, 0)))
    else:
        out_shape = jax.ShapeDtypeStruct((NPAD, dout), jnp.float32)
        out_specs = pl.BlockSpec((BN, dout), lambda t, i: (i, 0))

    args = [p.reshape(NCORES, NPAD, p.shape[1]) for p in parts]
    return pl.pallas_call(
        body,
        grid=(2, NBLK),
        in_specs=in_specs,
        out_specs=out_specs,
        out_shape=out_shape,
        scratch_shapes=[pltpu.VMEM((1, cin), jnp.float32),
                        pltpu.VMEM((cin, cin), jnp.float32)],
    )(*args, W.astype(jnp.float32), g.reshape(1, dout).astype(jnp.float32),
      be.reshape(1, dout).astype(jnp.float32))


def _fc_head(sums, cnts, p):
    def body(s_ref, c_ref, w1, b1, w2, b2, w3, b3, out_ref):
        s = s_ref[0] + s_ref[1]
        c = c_ref[0][:, 0:1] + c_ref[1][:, 0:1]
        mean = s / jnp.maximum(c, 1.0)
        h = jnp.dot(mean, w1[...], preferred_element_type=jnp.float32) + b1[...]
        h = jnp.where(h >= 0, h, 0.01 * h)
        h = jnp.dot(h, w2[...], preferred_element_type=jnp.float32) + b2[...]
        h = jnp.where(h >= 0, h, 0.01 * h)
        h = jnp.dot(h, w3[...], preferred_element_type=jnp.float32) + b3[...]
        out_ref[...] = h

    return pl.pallas_call(
        body,
        out_shape=jax.ShapeDtypeStruct((G, 2), jnp.float32),
    )(sums.reshape(NCORES, G, 128), cnts.reshape(NCORES, G, 16),
      p["Wf1"], p["bf1"].reshape(1, 30),
      p["Wf2"], p["bf2"].reshape(1, 20),
      p["Wf3"], p["bf3"].reshape(1, 2))


def kernel(x, edge_index, edge_weight, batch, params):
    src = edge_index[0].astype(jnp.int32)
    dst = edge_index[1].astype(jnp.int32)
    ew = edge_weight.astype(jnp.float32)

    npad_e = EPAD - E
    spread = (jnp.arange(npad_e, dtype=jnp.int32) * 61) % N
    src2 = jnp.concatenate([src, spread]).reshape(IDXROWS, 128)
    dst2 = jnp.concatenate([dst, spread]).reshape(IDXROWS, 128)
    ew2 = jnp.concatenate(
        [ew, jnp.zeros((npad_e,), jnp.float32)]).reshape(IDXROWS, 128)

    xp = jnp.pad(x.astype(jnp.float32), ((0, NPAD - N), (0, 16 - x.shape[1])))
    b2 = jnp.pad(batch.astype(jnp.int32), (0, NPAD - N),
                 constant_values=G).reshape(HROWS, 128)

    p = params
    w1 = jnp.pad(p["W1"].astype(jnp.float32),
                 ((0, 16 - p["W1"].shape[0]), (0, 0)))

    sc16 = _make_edge_scatter(16)
    sc32 = _make_edge_scatter(32)

    acc1 = sc16(xp, src2, dst2, ew2)
    h1 = _tc_layer([acc1], w1, p["g1"], p["be1"])

    acc2 = sc16(h1, src2, dst2, ew2)
    h2 = _tc_layer([acc2], p["W2"], p["g2"], p["be2"])

    acc3 = sc16(h2, src2, dst2, ew2)
    h3 = _tc_layer([acc3], p["W3"], p["g3"], p["be3"])

    acc4 = sc32(h3, src2, dst2, ew2)
    h4a, h4b = _tc_layer([acc4], p["W4"], p["g4"], p["be4"], split=True)

    acc5a = sc32(h4a, src2, dst2, ew2)
    acc5b = sc32(h4b, src2, dst2, ew2)
    h5 = _tc_layer([acc5a, acc5b], p["W5"], p["g5"], p["be5"])

    sums, cnts = _pool_kernel(h5, b2)
    return _fc_head(sums, cnts, p)


# speedup vs baseline: 7.7744x; 7.7744x over previous
import functools

import jax
import jax.numpy as jnp
from jax import lax
from jax.experimental import pallas as pl
from jax.experimental.pallas import tpu as pltpu
from jax.experimental.pallas import tpu_sc as plsc

N = 50000
NPAD = 50048
E = 800000
EPAD = 819200
G = 1024
GACC = 1040
NCORES = 2
NSUB = 16
NW = NCORES * NSUB

IDXROWS = EPAD // 128
IDXROWS_PER_W = IDXROWS // NW
RPT = NPAD // NSUB
ZROWS = RPT // 8

HROWS = NPAD // 128
BN = 544
NBLK = NPAD // BN


def _mesh():
    return plsc.VectorSubcoreMesh(
        core_axis_name="c", subcore_axis_name="s",
        num_cores=NCORES, num_subcores=NSUB)


@functools.lru_cache(maxsize=None)
def _make_edge_scatter(C):
    vpr = C // 16
    chunk = 16384 // C
    subrows = chunk // 128
    nchunks = IDXROWS_PER_W // subrows

    @functools.partial(
        pl.kernel,
        out_type=jax.ShapeDtypeStruct((NCORES * NPAD, C), jnp.float32),
        mesh=_mesh(),
        scratch_types=[
            pltpu.VMEM((subrows, 128), jnp.int32),
            pltpu.VMEM((subrows, 128), jnp.int32),
            pltpu.VMEM((chunk,), jnp.float32),
            pltpu.VMEM((chunk, C), jnp.float32),
            pltpu.VMEM_SHARED((NPAD, C), jnp.float32),
            pltpu.SemaphoreType.DMA,
        ],
        compiler_params=pltpu.CompilerParams(use_tc_tiling_on_sc=False),
    )
    def edge_scatter(z_hbm, src_hbm, dst_hbm, ew_hbm, out_hbm,
                     src_v, dst_v, ew_v, rows_v, acc_sh, sem):
        cid = lax.axis_index("c")
        sid = lax.axis_index("s")
        wid = cid * NSUB + sid

        @pl.loop(0, ZROWS)
        def _zero_fill(r):
            for v in range(vpr):
                rows_v[r, pl.ds(16 * v, 16)] = jnp.zeros((16,), jnp.float32)

        for q in range(8):
            pltpu.sync_copy(
                rows_v.at[pl.ds(0, ZROWS)],
                acc_sh.at[pl.ds(sid * RPT + q * ZROWS, ZROWS)])
        plsc.subcore_barrier()

        base_row = wid * IDXROWS_PER_W

        @pl.loop(0, nchunks)
        def _chunk(gidx):
            r0 = base_row + gidx * subrows
            pltpu.sync_copy(src_hbm.at[pl.ds(r0, subrows)], src_v)
            pltpu.sync_copy(dst_hbm.at[pl.ds(r0, subrows)], dst_v)
            pltpu.sync_copy(ew_hbm.at[pl.ds(r0 * 128, chunk)], ew_v)
            gathers = [
                pltpu.async_copy(z_hbm.at[src_v.at[j]],
                                 rows_v.at[pl.ds(j * 128, 128)], sem)
                for j in range(subrows)
            ]
            for cp in gathers:
                cp.wait()

            @pl.loop(0, chunk // 16)
            def _scale(gq):
                wv = ew_v[pl.ds(gq * 16, 16)]
                dnums = lax.GatherDimensionNumbers(
                    offset_dims=(), collapsed_slice_dims=(0,),
                    start_index_map=(0,))
                for i in range(16):
                    w = lax.gather(
                        wv, jnp.full((16, 1), i, jnp.int32), dnums,
                        slice_sizes=(1,),
                        mode=lax.GatherScatterMode.PROMISE_IN_BOUNDS)
                    e = gq * 16 + i
                    for v in range(vpr):
                        rows_v[e, pl.ds(16 * v, 16)] = (
                            rows_v[e, pl.ds(16 * v, 16)] * w)

            scatters = [
                pltpu.async_copy(rows_v.at[pl.ds(j * 128, 128)],
                                 acc_sh.at[dst_v.at[j]], sem, add=True)
                for j in range(subrows)
            ]
            for cp in scatters:
                cp.wait()

        plsc.subcore_barrier()
        pltpu.sync_copy(
            acc_sh.at[pl.ds(sid * RPT, RPT)],
            out_hbm.at[pl.ds(cid * NPAD + sid * RPT, RPT)])

    return edge_scatter


@functools.partial(
    pl.kernel,
    out_type=(jax.ShapeDtypeStruct((NCORES * G, 128), jnp.float32),
              jax.ShapeDtypeStruct((NCORES * G, 16), jnp.float32)),
    mesh=_mesh(),
    scratch_types=[
        pltpu.VMEM((1, 128), jnp.int32),
        pltpu.VMEM((128, 128), jnp.float32),
        pltpu.VMEM((128, 16), jnp.float32),
        pltpu.VMEM((65, 128), jnp.float32),
        pltpu.VMEM((65, 16), jnp.float32),
        pltpu.VMEM_SHARED((GACC, 128), jnp.float32),
        pltpu.VMEM_SHARED((GACC, 16), jnp.float32),
    ],
    compiler_params=pltpu.CompilerParams(use_tc_tiling_on_sc=False),
)
def _pool_kernel(h_hbm, b_hbm, sum_hbm, cnt_hbm,
                 bidx_v, rows_v, ones_v, zs_v, zc_v, accs_sh, accc_sh):
    cid = lax.axis_index("c")
    sid = lax.axis_index("s")
    wid = cid * NSUB + sid

    @pl.loop(0, 65)
    def _fill_z(r):
        for v in range(8):
            zs_v[r, pl.ds(16 * v, 16)] = jnp.zeros((16,), jnp.float32)
        zc_v[r, pl.ds(0, 16)] = jnp.zeros((16,), jnp.float32)

    @pl.loop(0, 128)
    def _fill_one(r):
        ones_v[r, pl.ds(0, 16)] = jnp.ones((16,), jnp.float32)

    pltpu.sync_copy(zs_v, accs_sh.at[pl.ds(sid * 65, 65)])
    pltpu.sync_copy(zc_v, accc_sh.at[pl.ds(sid * 65, 65)])
    plsc.subcore_barrier()

    extra = jnp.where(wid < 7, 1, 0)
    base = wid * 12 + jnp.minimum(wid, 7)

    @pl.loop(0, 12 + extra)
    def _row(jj):
        j = base + jj
        pltpu.sync_copy(h_hbm.at[pl.ds(j * 128, 128)], rows_v)
        pltpu.sync_copy(b_hbm.at[pl.ds(j, 1)], bidx_v)
        pltpu.sync_copy(rows_v, accs_sh.at[bidx_v.at[0]], add=True)
        pltpu.sync_copy(ones_v, accc_sh.at[bidx_v.at[0]], add=True)

    plsc.subcore_barrier()
    pltpu.sync_copy(accs_sh.at[pl.ds(sid * 64, 64)],
                    sum_hbm.at[pl.ds(cid * G + sid * 64, 64)])
    pltpu.sync_copy(accc_sh.at[pl.ds(sid * 64, 64)],
                    cnt_hbm.at[pl.ds(cid * G + sid * 64, 64)])


def _tc_layer(parts, W, g, be, split=False):
    widths = [p.shape[1] for p in parts]
    cin = sum(widths)
    dout = W.shape[1]
    nparts = len(parts)

    def body(*refs):
        a_refs = refs[:2 * nparts]
        w_ref, g_ref, be_ref = refs[2 * nparts:2 * nparts + 3]
        if split:
            outa, outb, sacc, c2acc = refs[2 * nparts + 3:]
        else:
            out, sacc, c2acc = refs[2 * nparts + 3:]
        t = pl.program_id(0)
        i = pl.program_id(1)

        zs = []
        for p in range(nparts):
            zs.append(a_refs[2 * p][0] + a_refs[2 * p + 1][0])
        z = zs[0] if nparts == 1 else jnp.concatenate(zs, axis=1)

        @pl.when(jnp.logical_and(t == 0, i == 0))
        def _init():
            sacc[...] = jnp.zeros_like(sacc)
            c2acc[...] = jnp.zeros_like(c2acc)

        @pl.when(t == 0)
        def _stats():
            sacc[...] += jnp.sum(z, axis=0, keepdims=True)
            c2acc[...] += lax.dot_general(
                z, z, (((0,), (0,)), ((), ())),
                preferred_element_type=jnp.float32,
                precision=lax.Precision.HIGHEST)

        @pl.when(t == 1)
        def _emit():
            wmat = w_ref[...]
            m = sacc[...] * (1.0 / N)
            my = jnp.dot(m, wmat, preferred_element_type=jnp.float32, precision=lax.Precision.HIGHEST)
            p2 = jnp.dot(c2acc[...] * (1.0 / N), wmat,
                         preferred_element_type=jnp.float32, precision=lax.Precision.HIGHEST)
            ey2 = jnp.sum(wmat * p2, axis=0, keepdims=True)
            var = ey2 - my * my
            scale = g_ref[...] * lax.rsqrt(var + 1e-5)
            shift = be_ref[...] - my * scale
            y = jnp.dot(z, wmat,
                        preferred_element_type=jnp.float32, precision=lax.Precision.HIGHEST) * scale + shift
            y = jnp.where(y >= 0, y, 0.01 * y)
            if split:
                outa[...] = y[:, :dout // 2]
                outb[...] = y[:, dout // 2:]
            else:
                out[...] = y

    in_specs = []
    for p, cp in enumerate(widths):
        for c in range(NCORES):
            in_specs.append(pl.BlockSpec(
                (1, BN, cp), lambda t, i, c=c: (c, i, 0)))
    in_specs += [
        pl.BlockSpec((cin, dout), lambda t, i: (0, 0)),
        pl.BlockSpec((1, dout), lambda t, i: (0, 0)),
        pl.BlockSpec((1, dout), lambda t, i: (0, 0)),
    ]
    if split:
        out_shape = (jax.ShapeDtypeStruct((NPAD, dout // 2), jnp.float32),
                     jax.ShapeDtypeStruct((NPAD, dout // 2), jnp.float32))
        out_specs = (pl.BlockSpec((BN, dout // 2), lambda t, i: (i, 0)),
                     pl.BlockSpec((BN, dout // 2), lambda t, i: (i, 0)))
    else:
        out_shape = jax.ShapeDtypeStruct((NPAD, dout), jnp.float32)
        out_specs = pl.BlockSpec((BN, dout), lambda t, i: (i, 0))

    args = []
    for p in parts:
        p3 = p.reshape(NCORES, NPAD, p.shape[1])
        args += [p3, p3]
    return pl.pallas_call(
        body,
        grid=(2, NBLK),
        in_specs=in_specs,
        out_specs=out_specs,
        out_shape=out_shape,
        scratch_shapes=[pltpu.VMEM((1, cin), jnp.float32),
                        pltpu.VMEM((cin, cin), jnp.float32)],
    )(*args, W.astype(jnp.float32), g.reshape(1, dout).astype(jnp.float32),
      be.reshape(1, dout).astype(jnp.float32))


def _fc_head(sums, cnts, p):
    def body(s_ref, c_ref, w1, b1, w2, b2, w3, b3, out_ref):
        s = s_ref[0] + s_ref[1]
        c = c_ref[0][:, 0:1] + c_ref[1][:, 0:1]
        mean = s / jnp.maximum(c, 1.0)
        h = jnp.dot(mean, w1[...], preferred_element_type=jnp.float32, precision=lax.Precision.HIGHEST) + b1[...]
        h = jnp.where(h >= 0, h, 0.01 * h)
        h = jnp.dot(h, w2[...], preferred_element_type=jnp.float32, precision=lax.Precision.HIGHEST) + b2[...]
        h = jnp.where(h >= 0, h, 0.01 * h)
        h = jnp.dot(h, w3[...], preferred_element_type=jnp.float32, precision=lax.Precision.HIGHEST) + b3[...]
        out_ref[...] = h

    return pl.pallas_call(
        body,
        out_shape=jax.ShapeDtypeStruct((G, 2), jnp.float32),
    )(sums.reshape(NCORES, G, 128), cnts.reshape(NCORES, G, 16),
      p["Wf1"], p["bf1"].reshape(1, 30),
      p["Wf2"], p["bf2"].reshape(1, 20),
      p["Wf3"], p["bf3"].reshape(1, 2))


def kernel(x, edge_index, edge_weight, batch, params):
    src = edge_index[0].astype(jnp.int32)
    dst = edge_index[1].astype(jnp.int32)
    ew = edge_weight.astype(jnp.float32)

    npad_e = EPAD - E
    spread = (jnp.arange(npad_e, dtype=jnp.int32) * 61) % N
    src2 = jnp.concatenate([src, spread]).reshape(IDXROWS, 128)
    dst2 = jnp.concatenate([dst, spread]).reshape(IDXROWS, 128)
    ew2 = jnp.concatenate([ew, jnp.zeros((npad_e,), jnp.float32)])

    xp = jnp.pad(x.astype(jnp.float32), ((0, NPAD - N), (0, 16 - x.shape[1])))
    b2 = jnp.pad(batch.astype(jnp.int32), (0, NPAD - N),
                 constant_values=G).reshape(HROWS, 128)

    p = params
    w1 = jnp.pad(p["W1"].astype(jnp.float32),
                 ((0, 16 - p["W1"].shape[0]), (0, 0)))

    sc16 = _make_edge_scatter(16)
    sc32 = _make_edge_scatter(32)

    acc1 = sc16(xp, src2, dst2, ew2)
    h1 = _tc_layer([acc1], w1, p["g1"], p["be1"])

    acc2 = sc16(h1, src2, dst2, ew2)
    h2 = _tc_layer([acc2], p["W2"], p["g2"], p["be2"])

    acc3 = sc16(h2, src2, dst2, ew2)
    h3 = _tc_layer([acc3], p["W3"], p["g3"], p["be3"])

    acc4 = sc32(h3, src2, dst2, ew2)
    h4a, h4b = _tc_layer([acc4], p["W4"], p["g4"], p["be4"], split=True)

    acc5a = sc32(h4a, src2, dst2, ew2)
    acc5b = sc32(h4b, src2, dst2, ew2)
    h5 = _tc_layer([acc5a, acc5b], p["W5"], p["g5"], p["be5"])

    sums, cnts = _pool_kernel(h5, b2)
    return _fc_head(sums, cnts, p)


# speedup vs baseline: 9.2545x; 1.1904x over previous
import functools

import jax
import jax.numpy as jnp
from jax import lax
from jax.experimental import pallas as pl
from jax.experimental.pallas import tpu as pltpu
from jax.experimental.pallas import tpu_sc as plsc

N = 50000
NPAD = 50048
E = 800000
EPAD = 819200
G = 1024
GACC = 1040
NCORES = 2
NSUB = 16
NW = NCORES * NSUB

IDXROWS = EPAD // 128
IDXROWS_PER_W = IDXROWS // NW
RPT = NPAD // NSUB
ZROWS = RPT // 8

HROWS = NPAD // 128
BN = 544
NBLK = NPAD // BN


def _mesh():
    return plsc.VectorSubcoreMesh(
        core_axis_name="c", subcore_axis_name="s",
        num_cores=NCORES, num_subcores=NSUB)


CHUNK = 1280
SUBR = CHUNK // 128


@functools.lru_cache(maxsize=None)
def _make_edge_scatter(colsplit):
    zrows = 2 * NPAD if colsplit else NPAD
    rows_per_tile = IDXROWS // NSUB if colsplit else IDXROWS_PER_W
    nchunks = rows_per_tile // SUBR

    @functools.partial(
        pl.kernel,
        out_type=jax.ShapeDtypeStruct((NCORES * NPAD, 16), jnp.float32),
        mesh=_mesh(),
        scratch_types=[
            pltpu.VMEM((3 * SUBR, 128), jnp.int32),
            pltpu.VMEM((3 * SUBR, 128), jnp.int32),
            pltpu.VMEM((3 * CHUNK,), jnp.float32),
            pltpu.VMEM((3 * CHUNK, 16), jnp.float32),
            pltpu.VMEM_SHARED((NPAD, 16), jnp.float32),
            pltpu.SemaphoreType.DMA,
            pltpu.SemaphoreType.DMA,
            pltpu.SemaphoreType.DMA,
            pltpu.SemaphoreType.DMA,
            pltpu.SemaphoreType.DMA,
            pltpu.SemaphoreType.DMA,
        ],
        compiler_params=pltpu.CompilerParams(use_tc_tiling_on_sc=False),
    )
    def edge_scatter(z_hbm, src_hbm, dst_hbm, ew_hbm, out_hbm,
                     src_v, dst_v, ew_v, rows_v, acc_sh,
                     sg0, sg1, sg2, ss0, ss1, ss2):
        cid = lax.axis_index("c")
        sid = lax.axis_index("s")
        semg = [sg0, sg1, sg2]
        sems = [ss0, ss1, ss2]

        @pl.loop(0, ZROWS)
        def _zero_fill(r):
            rows_v[r, pl.ds(0, 16)] = jnp.zeros((16,), jnp.float32)

        for q in range(8):
            pltpu.sync_copy(
                rows_v.at[pl.ds(0, ZROWS)],
                acc_sh.at[pl.ds(sid * RPT + q * ZROWS, ZROWS)])
        plsc.subcore_barrier()

        if colsplit:
            base_row = sid * rows_per_tile
            zoff = cid * NPAD
        else:
            base_row = (cid * NSUB + sid) * rows_per_tile
            zoff = None

        def idx_load(i, p):
            r0 = base_row + i * SUBR
            pltpu.sync_copy(src_hbm.at[pl.ds(r0, SUBR)],
                            src_v.at[pl.ds(p * SUBR, SUBR)])
            pltpu.sync_copy(dst_hbm.at[pl.ds(r0, SUBR)],
                            dst_v.at[pl.ds(p * SUBR, SUBR)])
            pltpu.sync_copy(ew_hbm.at[pl.ds(r0 * 128, CHUNK)],
                            ew_v.at[pl.ds(p * CHUNK, CHUNK)])
            if colsplit:
                off = jnp.full((16,), zoff, jnp.int32)
                for j in range(SUBR):
                    for k in range(8):
                        src_v[p * SUBR + j, pl.ds(16 * k, 16)] = (
                            src_v[p * SUBR + j, pl.ds(16 * k, 16)] + off)

        def fire_g(p):
            for j in range(SUBR):
                pltpu.async_copy(
                    z_hbm.at[src_v.at[p * SUBR + j]],
                    rows_v.at[pl.ds(p * CHUNK + j * 128, 128)], semg[p])

        def wait_g(p):
            for j in range(SUBR):
                pltpu.make_async_copy(
                    z_hbm.at[src_v.at[p * SUBR + j]],
                    rows_v.at[pl.ds(p * CHUNK + j * 128, 128)],
                    semg[p]).wait()

        def fire_s(p):
            for j in range(SUBR):
                pltpu.async_copy(
                    rows_v.at[pl.ds(p * CHUNK + j * 128, 128)],
                    acc_sh.at[dst_v.at[p * SUBR + j]], sems[p], add=True)

        def wait_s(p):
            for j in range(SUBR):
                pltpu.make_async_copy(
                    rows_v.at[pl.ds(p * CHUNK + j * 128, 128)],
                    acc_sh.at[dst_v.at[p * SUBR + j]], sems[p]).wait()

        dnums = lax.GatherDimensionNumbers(
            offset_dims=(), collapsed_slice_dims=(0,), start_index_map=(0,))

        def scale(p):
            @pl.loop(0, CHUNK // 16)
            def _s(gq):
                e0 = p * CHUNK + gq * 16
                wv = ew_v[pl.ds(e0, 16)]
                for i in range(16):
                    w = lax.gather(
                        wv, jnp.full((16, 1), i, jnp.int32), dnums,
                        slice_sizes=(1,),
                        mode=lax.GatherScatterMode.PROMISE_IN_BOUNDS)
                    rows_v[e0 + i, pl.ds(0, 16)] = (
                        rows_v[e0 + i, pl.ds(0, 16)] * w)

        def stepA(i, p, with_wait):
            if with_wait:
                wait_s(p)
            idx_load(i, p)
            fire_g(p)

        def stepB(pm1):
            wait_g(pm1)
            scale(pm1)
            fire_s(pm1)

        stepA(0, 0, False)
        stepA(1, 1, False)
        stepB(0)
        stepA(2, 2, False)
        stepB(1)

        q_steady = (nchunks - 3) // 3
        rem = (nchunks - 3) % 3

        @pl.loop(0, q_steady)
        def _steady(qi):
            for b in range(3):
                i = 3 + 3 * qi + b
                stepA(i, b, True)
                stepB((b + 2) % 3)

        for i in range(nchunks - rem, nchunks):
            stepA(i, i % 3, True)
            stepB((i + 2) % 3)

        stepB((nchunks - 1) % 3)
        for p in range(3):
            wait_s(p)

        plsc.subcore_barrier()
        pltpu.sync_copy(
            acc_sh.at[pl.ds(sid * RPT, RPT)],
            out_hbm.at[pl.ds(cid * NPAD + sid * RPT, RPT)])

    return edge_scatter


@functools.partial(
    pl.kernel,
    out_type=(jax.ShapeDtypeStruct((NCORES * G, 128), jnp.float32),
              jax.ShapeDtypeStruct((NCORES * G, 16), jnp.float32)),
    mesh=_mesh(),
    scratch_types=[
        pltpu.VMEM((1, 128), jnp.int32),
        pltpu.VMEM((128, 128), jnp.float32),
        pltpu.VMEM((128, 16), jnp.float32),
        pltpu.VMEM((65, 128), jnp.float32),
        pltpu.VMEM((65, 16), jnp.float32),
        pltpu.VMEM_SHARED((GACC, 128), jnp.float32),
        pltpu.VMEM_SHARED((GACC, 16), jnp.float32),
    ],
    compiler_params=pltpu.CompilerParams(use_tc_tiling_on_sc=False),
)
def _pool_kernel(h_hbm, b_hbm, sum_hbm, cnt_hbm,
                 bidx_v, rows_v, ones_v, zs_v, zc_v, accs_sh, accc_sh):
    cid = lax.axis_index("c")
    sid = lax.axis_index("s")
    wid = cid * NSUB + sid

    @pl.loop(0, 65)
    def _fill_z(r):
        for v in range(8):
            zs_v[r, pl.ds(16 * v, 16)] = jnp.zeros((16,), jnp.float32)
        zc_v[r, pl.ds(0, 16)] = jnp.zeros((16,), jnp.float32)

    @pl.loop(0, 128)
    def _fill_one(r):
        ones_v[r, pl.ds(0, 16)] = jnp.ones((16,), jnp.float32)

    pltpu.sync_copy(zs_v, accs_sh.at[pl.ds(sid * 65, 65)])
    pltpu.sync_copy(zc_v, accc_sh.at[pl.ds(sid * 65, 65)])
    plsc.subcore_barrier()

    extra = jnp.where(wid < 7, 1, 0)
    base = wid * 12 + jnp.minimum(wid, 7)

    @pl.loop(0, 12 + extra)
    def _row(jj):
        j = base + jj
        pltpu.sync_copy(h_hbm.at[pl.ds(j * 128, 128)], rows_v)
        pltpu.sync_copy(b_hbm.at[pl.ds(j, 1)], bidx_v)
        pltpu.sync_copy(rows_v, accs_sh.at[bidx_v.at[0]], add=True)
        pltpu.sync_copy(ones_v, accc_sh.at[bidx_v.at[0]], add=True)

    plsc.subcore_barrier()
    pltpu.sync_copy(accs_sh.at[pl.ds(sid * 64, 64)],
                    sum_hbm.at[pl.ds(cid * G + sid * 64, 64)])
    pltpu.sync_copy(accc_sh.at[pl.ds(sid * 64, 64)],
                    cnt_hbm.at[pl.ds(cid * G + sid * 64, 64)])


def _tc_layer(parts, W, g, be, stacked_outs=0, round_w=True, round_out=True):
    widths = [arr.shape[2] * (2 if mode == "cols" else 1)
              for mode, arr in parts]
    cin = sum(widths)
    dout = W.shape[1]
    nparts = len(parts)

    def body(*refs):
        a_refs = refs[:2 * nparts]
        w_ref, g_ref, be_ref = refs[2 * nparts:2 * nparts + 3]
        if stacked_outs == 4:
            outa, outb, sacc, c2acc = refs[2 * nparts + 3:]
        else:
            out, sacc, c2acc = refs[2 * nparts + 3:]
        t = pl.program_id(0)
        i = pl.program_id(1)

        zs = []
        for p, (mode, _) in enumerate(parts):
            r0, r1 = a_refs[2 * p][0], a_refs[2 * p + 1][0]
            if mode == "pair":
                zs.append(r0 + r1)
            else:
                zs.append(r0)
                zs.append(r1)
        z = zs[0] if len(zs) == 1 else jnp.concatenate(zs, axis=1)

        @pl.when(jnp.logical_and(t == 0, i == 0))
        def _init():
            sacc[...] = jnp.zeros_like(sacc)
            c2acc[...] = jnp.zeros_like(c2acc)

        @pl.when(t == 0)
        def _stats():
            sacc[...] += jnp.sum(z, axis=0, keepdims=True)
            c2acc[...] += lax.dot_general(
                z, z, (((0,), (0,)), ((), ())),
                preferred_element_type=jnp.float32,
                precision=lax.Precision.HIGHEST)

        @pl.when(t == 1)
        def _emit():
            wmat = w_ref[...]
            if round_w:
                wmat = wmat.astype(jnp.bfloat16).astype(jnp.float32)
            m = sacc[...] * (1.0 / N)
            my = jnp.dot(m, wmat, preferred_element_type=jnp.float32, precision=lax.Precision.HIGHEST)
            p2 = jnp.dot(c2acc[...] * (1.0 / N), wmat,
                         preferred_element_type=jnp.float32, precision=lax.Precision.HIGHEST)
            ey2 = jnp.sum(wmat * p2, axis=0, keepdims=True)
            var = ey2 - my * my
            scale = g_ref[...] * lax.rsqrt(var + 1e-5)
            shift = be_ref[...] - my * scale
            y = jnp.dot(z, wmat,
                        preferred_element_type=jnp.float32, precision=lax.Precision.HIGHEST) * scale + shift
            y = jnp.where(y >= 0, y, 0.01 * y)
            if round_out:
                y = y.astype(jnp.bfloat16).astype(jnp.float32)
            if stacked_outs == 4:
                w4 = dout // 4
                outa[0] = y[:, 0 * w4:1 * w4]
                outa[1] = y[:, 1 * w4:2 * w4]
                outb[0] = y[:, 2 * w4:3 * w4]
                outb[1] = y[:, 3 * w4:4 * w4]
            elif stacked_outs == 2:
                w2 = dout // 2
                out[0] = y[:, :w2]
                out[1] = y[:, w2:]
            else:
                out[...] = y

    in_specs = []
    for p, (mode, arr) in enumerate(parts):
        cp = arr.shape[2]
        for c in range(NCORES):
            in_specs.append(pl.BlockSpec(
                (1, BN, cp), lambda t, i, c=c: (c, i, 0)))
    in_specs += [
        pl.BlockSpec((cin, dout), lambda t, i: (0, 0)),
        pl.BlockSpec((1, dout), lambda t, i: (0, 0)),
        pl.BlockSpec((1, dout), lambda t, i: (0, 0)),
    ]
    if stacked_outs == 4:
        w4 = dout // 4
        out_shape = (jax.ShapeDtypeStruct((2, NPAD, w4), jnp.float32),
                     jax.ShapeDtypeStruct((2, NPAD, w4), jnp.float32))
        out_specs = (pl.BlockSpec((2, BN, w4), lambda t, i: (0, i, 0)),
                     pl.BlockSpec((2, BN, w4), lambda t, i: (0, i, 0)))
    elif stacked_outs == 2:
        w2 = dout // 2
        out_shape = jax.ShapeDtypeStruct((2, NPAD, w2), jnp.float32)
        out_specs = pl.BlockSpec((2, BN, w2), lambda t, i: (0, i, 0))
    else:
        out_shape = jax.ShapeDtypeStruct((NPAD, dout), jnp.float32)
        out_specs = pl.BlockSpec((BN, dout), lambda t, i: (i, 0))

    args = []
    for mode, arr in parts:
        args += [arr, arr]
    return pl.pallas_call(
        body,
        grid=(2, NBLK),
        in_specs=in_specs,
        out_specs=out_specs,
        out_shape=out_shape,
        scratch_shapes=[pltpu.VMEM((1, cin), jnp.float32),
                        pltpu.VMEM((cin, cin), jnp.float32)],
    )(*args, W.astype(jnp.float32), g.reshape(1, dout).astype(jnp.float32),
      be.reshape(1, dout).astype(jnp.float32))


def _fc_head(sums, cnts, p):
    def body(s_ref, c_ref, w1, b1, w2, b2, w3, b3, out_ref):
        s = s_ref[0] + s_ref[1]
        c = c_ref[0][:, 0:1] + c_ref[1][:, 0:1]
        mean = s / jnp.maximum(c, 1.0)
        h = jnp.dot(mean, w1[...], preferred_element_type=jnp.float32, precision=lax.Precision.HIGHEST) + b1[...]
        h = jnp.where(h >= 0, h, 0.01 * h)
        h = jnp.dot(h, w2[...], preferred_element_type=jnp.float32, precision=lax.Precision.HIGHEST) + b2[...]
        h = jnp.where(h >= 0, h, 0.01 * h)
        h = jnp.dot(h, w3[...], preferred_element_type=jnp.float32, precision=lax.Precision.HIGHEST) + b3[...]
        out_ref[...] = h

    return pl.pallas_call(
        body,
        out_shape=jax.ShapeDtypeStruct((G, 2), jnp.float32),
    )(sums.reshape(NCORES, G, 128), cnts.reshape(NCORES, G, 16),
      p["Wf1"], p["bf1"].reshape(1, 30),
      p["Wf2"], p["bf2"].reshape(1, 20),
      p["Wf3"], p["bf3"].reshape(1, 2))


def kernel(x, edge_index, edge_weight, batch, params):
    src = edge_index[0].astype(jnp.int32)
    dst = edge_index[1].astype(jnp.int32)
    ew = edge_weight.astype(jnp.float32)

    npad_e = EPAD - E
    spread = (jnp.arange(npad_e, dtype=jnp.int32) * 61) % N
    src2 = jnp.concatenate([src, spread]).reshape(IDXROWS, 128)
    dst2 = jnp.concatenate([dst, spread]).reshape(IDXROWS, 128)
    ew2 = jnp.concatenate([ew, jnp.zeros((npad_e,), jnp.float32)])

    xp = jnp.pad(x.astype(jnp.float32), ((0, NPAD - N), (0, 16 - x.shape[1])))
    b2 = jnp.pad(batch.astype(jnp.int32), (0, NPAD - N),
                 constant_values=G).reshape(HROWS, 128)

    p = params
    w1 = jnp.pad(p["W1"].astype(jnp.float32),
                 ((0, 16 - p["W1"].shape[0]), (0, 0)))

    sc_pair = _make_edge_scatter(False)
    sc_cols = _make_edge_scatter(True)

    def pair(a):
        return ("pair", a.reshape(NCORES, NPAD, 16))

    def cols(a):
        return ("cols", a.reshape(NCORES, NPAD, 16))

    acc1 = sc_pair(xp, src2, dst2, ew2)
    h1 = _tc_layer([pair(acc1)], w1, p["g1"], p["be1"], round_w=False)

    acc2 = sc_pair(h1, src2, dst2, ew2)
    h2 = _tc_layer([pair(acc2)], p["W2"], p["g2"], p["be2"])

    acc3 = sc_pair(h2, src2, dst2, ew2)
    h3 = _tc_layer([pair(acc3)], p["W3"], p["g3"], p["be3"], stacked_outs=2)

    acc4 = sc_cols(h3.reshape(2 * NPAD, 16), src2, dst2, ew2)
    h4a, h4b = _tc_layer([cols(acc4)], p["W4"], p["g4"], p["be4"],
                         stacked_outs=4)

    acc5a = sc_cols(h4a.reshape(2 * NPAD, 16), src2, dst2, ew2)
    acc5b = sc_cols(h4b.reshape(2 * NPAD, 16), src2, dst2, ew2)
    h5 = _tc_layer([cols(acc5a), cols(acc5b)], p["W5"], p["g5"], p["be5"],
                   round_out=False)

    sums, cnts = _pool_kernel(h5, b2)
    return _fc_head(sums, cnts, p)
